# baseline (speedup 1.0000x reference)
import functools

import jax
import jax.numpy as jnp
from jax import lax
from jax.experimental import pallas as pl
from jax.experimental.pallas import tpu as pltpu
from jax.experimental.pallas import tpu_sc as plsc


def _make_sc_embed(B, D, NC, NS, L):
    NW = NC * NS
    b_per_w = B // NW
    CH = 128
    n_ch = b_per_w // CH
    D2 = 2 * D

    mesh = plsc.VectorSubcoreMesh(core_axis_name="c", subcore_axis_name="s")

    @functools.partial(
        pl.kernel,
        mesh=mesh,
        out_type=jax.ShapeDtypeStruct((B // 2, D2), jnp.float32),
        scratch_types=[
            pltpu.VMEM((n_ch, CH), jnp.int32),
            pltpu.VMEM((n_ch, CH), jnp.int32),
            pltpu.VMEM((2, CH, D2), jnp.float32),
            pltpu.VMEM((2, CH // 2, D2), jnp.float32),
            pltpu.VMEM((L,), jnp.float32),
            pltpu.SemaphoreType.DMA((n_ch,)),
            pltpu.SemaphoreType.DMA((2,)),
        ],
    )
    def sc_embed(tok_hbm, scale_hbm, table_hbm, out_hbm,
                 idx_v, pidx_v, pair_v, outs_v, scale_v, gsems, ssems):
        wid = lax.axis_index("s") * NC + lax.axis_index("c")
        base = wid * (b_per_w // 2)
        pltpu.sync_copy(tok_hbm.at[wid], idx_v)
        pltpu.sync_copy(scale_hbm, scale_v)
        for c in range(n_ch):
            for k in range(CH // L):
                sl = (c, pl.ds(k * L, L))
                pidx_v[sl] = idx_v[sl] >> 1
        gathers = [None] * n_ch

        def fire(c):
            gathers[c] = pltpu.async_copy(
                table_hbm.at[pidx_v.at[c]], pair_v.at[c % 2], gsems.at[c])

        fire(0)
        fire(1)
        s = scale_v[...]
        store_h = [None, None]
        for c in range(n_ch):
            buf = c % 2
            gathers[c].wait()
            if store_h[buf] is not None:
                store_h[buf].wait()

            def g_body(g, carry, buf=buf, c=c):
                idx16 = idx_v[c, pl.ds(g * L, L)]
                h16 = (idx16 & 1) * D
                for l in range(L):
                    r = g * L + l
                    h = h16[l]
                    for j in range(D // L):
                        v = pair_v[buf, r, pl.ds(h + j * L, L)]
                        outs_v[buf, g * (L // 2) + l // 2,
                               pl.ds((l % 2) * D + j * L, L)] = v * s
                return carry

            lax.fori_loop(0, CH // L, g_body, 0)
            if c + 2 < n_ch:
                fire(c + 2)
            store_h[buf] = pltpu.async_copy(
                outs_v.at[buf],
                out_hbm.at[pl.ds(base + c * (CH // 2), CH // 2)],
                ssems.at[buf])
        store_h[0].wait()
        store_h[1].wait()

    return sc_embed


def kernel(token_ids, embed_weight, scale):
    B0, B1 = token_ids.shape
    V, D = embed_weight.shape
    B = B0 * B1
    info = plsc.get_sparse_core_info()
    NC, NS, L = info.num_cores, info.num_subcores, info.num_lanes
    NW = NC * NS
    b_per_w = B // NW
    CH = 128
    n_ch = b_per_w // CH

    tok = token_ids.reshape(NW, n_ch, CH).astype(jnp.int32)
    table2 = embed_weight.reshape(V // 2, 2 * D)
    scale16 = jnp.broadcast_to(scale.astype(jnp.float32).reshape(1), (L,))
    out2 = _make_sc_embed(B, D, NC, NS, L)(tok, scale16, table2)
    return out2.reshape(B0, B1, D)


# speedup vs baseline: 1.7349x; 1.7349x over previous
import functools

import jax
import jax.numpy as jnp
from jax import lax
from jax.experimental import pallas as pl
from jax.experimental.pallas import tpu as pltpu
from jax.experimental.pallas import tpu_sc as plsc


def _make_sc_embed(D, B, NC, NS, L):
    NW = NC * NS
    TPW = B // NW
    HTPW = TPW // 2
    FIRE = 16

    mesh = plsc.VectorSubcoreMesh(core_axis_name="c", subcore_axis_name="s")

    @functools.partial(
        pl.kernel,
        mesh=mesh,
        out_type=jax.ShapeDtypeStruct((B, D), jnp.float32),
        scratch_types=[
            pltpu.VMEM((TPW,), jnp.int32),
            pltpu.VMEM((HTPW, D), jnp.float32),
            pltpu.VMEM((L,), jnp.float32),
            pltpu.SemaphoreType.DMA,
            pltpu.SemaphoreType.DMA,
        ],
    )
    def sc_embed(tok_hbm, scale_hbm, table_hbm, out_hbm,
                 idx_v, stage, scale_v, gsem, ssem):
        wid = lax.axis_index("s") * NC + lax.axis_index("c")
        pltpu.sync_copy(tok_hbm.at[wid], idx_v)
        pltpu.sync_copy(scale_hbm, scale_v)
        s = scale_v[...]

        for p in range(2):
            def fire_body(g, carry, p=p):
                idx16 = idx_v[pl.ds(p * HTPW + g * FIRE, FIRE)]
                for k in range(FIRE):
                    lg = g * FIRE + k
                    i_t = idx16[k]
                    pltpu.async_copy(
                        table_hbm.at[i_t], stage.at[lg], gsem)
                return carry

            lax.fori_loop(0, HTPW // FIRE, fire_body, 0)
            pltpu.make_async_copy(
                table_hbm.at[pl.ds(0, HTPW)], stage, gsem).wait()

            def scale_body(r, carry):
                for k in range(D // L):
                    sl = (r, pl.ds(k * L, L))
                    stage[sl] = stage[sl] * s
                return carry

            lax.fori_loop(0, HTPW, scale_body, 0)
            pltpu.async_copy(
                stage,
                out_hbm.at[pl.ds(wid * TPW + p * HTPW, HTPW)],
                ssem).wait()

    return sc_embed


def kernel(token_ids, embed_weight, scale):
    B0, B1 = token_ids.shape
    V, D = embed_weight.shape
    B = B0 * B1
    info = plsc.get_sparse_core_info()
    NC, NS, L = info.num_cores, info.num_subcores, info.num_lanes
    NW = NC * NS

    tok = token_ids.reshape(NW, B // NW).astype(jnp.int32)
    scale16 = jnp.broadcast_to(scale.astype(jnp.float32).reshape(1), (L,))
    out = _make_sc_embed(D, B, NC, NS, L)(tok, scale16, embed_weight)
    return out.reshape(B0, B1, D)
